# Initial kernel scaffold; baseline (speedup 1.0000x reference)
import jax
import jax.numpy as jnp
from jax.experimental import pallas as pl


def kernel(X, ref_a, ref_b, W_hidden, b_hidden, W_out, b_out):
    raise NotImplementedError("write your pallas kernel here")


# speedup vs baseline: 4.6147x; 4.6147x over previous
import functools

import jax
import jax.numpy as jnp
from jax import lax
from jax.experimental import pallas as pl
from jax.experimental.pallas import tpu as pltpu
from jax.experimental.pallas import tpu_sc as plsc

N, E, D, H = 10000, 320000, 128, 128

NC, NS = 2, 16
NW = NC * NS
E2 = 2 * E
PER_W = E2 // NW
CHUNK = 80
ITERS = PER_W // CHUNK
R_MAIN = 632
R_LAST = N - (NS - 1) * R_MAIN


def _sc_aggregate():
    mesh = plsc.VectorSubcoreMesh(
        core_axis_name="c", subcore_axis_name="s", num_cores=NC, num_subcores=NS
    )

    @functools.partial(
        pl.kernel,
        out_type=jax.ShapeDtypeStruct((NC, N, D), jnp.float32),
        mesh=mesh,
        scratch_types=[
            pltpu.VMEM((CHUNK,), jnp.int32),
            pltpu.VMEM((CHUNK,), jnp.int32),
            pltpu.VMEM((CHUNK, D), jnp.float32),
            pltpu.VMEM_SHARED((N, D), jnp.float32),
            pltpu.SemaphoreType.DMA,
        ],
    )
    def sc_agg(x_hbm, src_hbm, dst_hbm, out_hbm, sidx, didx, rows, acc, sem):
        c = lax.axis_index("c")
        s = lax.axis_index("s")
        wid = c * NS + s

        @pl.when(s < NS - 1)
        def _():
            r0 = pl.multiple_of(s * R_MAIN, 8)
            pltpu.sync_copy(x_hbm.at[pl.ds(r0, R_MAIN)], acc.at[pl.ds(r0, R_MAIN)])

        @pl.when(s == NS - 1)
        def _():
            r0 = (NS - 1) * R_MAIN
            pltpu.sync_copy(x_hbm.at[pl.ds(r0, R_LAST)], acc.at[pl.ds(r0, R_LAST)])

        plsc.subcore_barrier()

        base = wid * PER_W

        def body(j, carry):
            off = base + j * CHUNK
            pltpu.sync_copy(src_hbm.at[pl.ds(off, CHUNK)], sidx)
            pltpu.sync_copy(dst_hbm.at[pl.ds(off, CHUNK)], didx)
            pltpu.async_copy(x_hbm.at[sidx], rows, sem).wait()
            pltpu.sync_copy(rows, acc.at[didx], add=True)
            return carry

        lax.fori_loop(0, ITERS, body, 0)
        plsc.subcore_barrier()

        @pl.when(s < NS - 1)
        def _():
            r0 = pl.multiple_of(s * R_MAIN, 8)
            pltpu.sync_copy(
                acc.at[pl.ds(r0, R_MAIN)], out_hbm.at[c, pl.ds(r0, R_MAIN)]
            )

        @pl.when(s == NS - 1)
        def _():
            r0 = (NS - 1) * R_MAIN
            pltpu.sync_copy(
                acc.at[pl.ds(r0, R_LAST)], out_hbm.at[c, pl.ds(r0, R_LAST)]
            )

    return sc_agg


_ROW_BLK = 1000


def _mlp_body(a0_ref, a1_ref, x_ref, wh_ref, bh_ref, wo_ref, bo_ref, o_ref):
    xa = a0_ref[...] + a1_ref[...] - x_ref[...]
    h = (
        jnp.dot(xa, wh_ref[...], preferred_element_type=jnp.float32,
                precision=lax.Precision.HIGHEST)
        + bh_ref[...]
    )
    o = (
        jnp.dot(h, wo_ref[...], preferred_element_type=jnp.float32,
                precision=lax.Precision.HIGHEST)
        + bo_ref[...]
    )
    o_ref[...] = jnp.maximum(o, 0.0)


def _tc_mlp(a0, a1, x, wh, bh, wo, bo):
    grid = (N // _ROW_BLK,)
    row_spec = pl.BlockSpec((_ROW_BLK, D), lambda i: (i, 0))
    full_w = pl.BlockSpec((D, H), lambda i: (0, 0))
    full_b = pl.BlockSpec((1, H), lambda i: (0, 0))
    return pl.pallas_call(
        _mlp_body,
        grid=grid,
        in_specs=[row_spec, row_spec, row_spec, full_w, full_b, full_w, full_b],
        out_specs=pl.BlockSpec((_ROW_BLK, H), lambda i: (i, 0)),
        out_shape=jax.ShapeDtypeStruct((N, H), jnp.float32),
    )(a0, a1, x, wh, bh, wo, bo)


@jax.jit
def kernel(X, ref_a, ref_b, W_hidden, b_hidden, W_out, b_out):
    src = jnp.concatenate([ref_a, ref_b])
    dst = jnp.concatenate([ref_b, ref_a])
    accs = _sc_aggregate()(X, src, dst)
    return _tc_mlp(
        accs[0], accs[1], X,
        W_hidden, b_hidden.reshape(1, H), W_out, b_out.reshape(1, H),
    )


# speedup vs baseline: 10.9342x; 2.3694x over previous
import functools

import jax
import jax.numpy as jnp
from jax import lax
from jax.experimental import pallas as pl
from jax.experimental.pallas import tpu as pltpu
from jax.experimental.pallas import tpu_sc as plsc

N, E, D, H = 10000, 320000, 128, 128

NC, NS = 2, 16
NW = NC * NS
E2 = 2 * E
PER_W = E2 // NW
CHUNK = 80
ITERS = PER_W // CHUNK
NBUF = 2
GROUPS = ITERS // NBUF
R_MAIN = 632
R_LAST = N - (NS - 1) * R_MAIN


def _sc_aggregate():
    mesh = plsc.VectorSubcoreMesh(
        core_axis_name="c", subcore_axis_name="s", num_cores=NC, num_subcores=NS
    )

    @functools.partial(
        pl.kernel,
        out_type=jax.ShapeDtypeStruct((NC, N, D), jnp.float32),
        mesh=mesh,
        scratch_types=(
            [pltpu.VMEM((PER_W,), jnp.int32)]
            + [pltpu.VMEM((CHUNK, D), jnp.float32) for _ in range(NBUF)]
            + [pltpu.VMEM((CHUNK,), jnp.int32) for _ in range(NBUF)]
            + [pltpu.VMEM((CHUNK,), jnp.int32) for _ in range(NBUF)]
            + [pltpu.SemaphoreType.DMA for _ in range(NBUF)]
            + [pltpu.VMEM_SHARED((N, D), jnp.float32)]
        ),
    )
    def sc_agg(x_hbm, pk_hbm, out_hbm, pk_v, *rest):
        rows = rest[:NBUF]
        sidx = rest[NBUF:2 * NBUF]
        didx = rest[2 * NBUF:3 * NBUF]
        sems = rest[3 * NBUF:4 * NBUF]
        acc = rest[4 * NBUF]
        c = lax.axis_index("c")
        s = lax.axis_index("s")
        wid = c * NS + s

        def unpack(j, b):
            for k in range(CHUNK // 16):
                v = pk_v[pl.ds(j * CHUNK + 16 * k, 16)]
                sidx[b][pl.ds(16 * k, 16)] = v & jnp.int32(0xFFFF)
                didx[b][pl.ds(16 * k, 16)] = lax.shift_right_logical(
                    v, jnp.int32(16)
                )

        @pl.when(s < NS - 1)
        def _():
            r0 = pl.multiple_of(s * R_MAIN, 8)
            pltpu.sync_copy(x_hbm.at[pl.ds(r0, R_MAIN)], acc.at[pl.ds(r0, R_MAIN)])

        @pl.when(s == NS - 1)
        def _():
            r0 = (NS - 1) * R_MAIN
            pltpu.sync_copy(x_hbm.at[pl.ds(r0, R_LAST)], acc.at[pl.ds(r0, R_LAST)])

        pltpu.sync_copy(pk_hbm.at[wid], pk_v)
        plsc.subcore_barrier()

        for b in range(NBUF):
            unpack(b, b)
            pltpu.async_copy(x_hbm.at[sidx[b]], rows[b], sems[b])

        def group(o, carry):
            for b in range(NBUF):
                j = o * NBUF + b
                pltpu.make_async_copy(x_hbm.at[sidx[b]], rows[b], sems[b]).wait()
                pltpu.sync_copy(rows[b], acc.at[didx[b]], add=True)

                @pl.when(j + NBUF < ITERS)
                def _():
                    unpack(j + NBUF, b)
                    pltpu.async_copy(x_hbm.at[sidx[b]], rows[b], sems[b])

            return carry

        lax.fori_loop(0, GROUPS, group, 0)
        plsc.subcore_barrier()

        @pl.when(s < NS - 1)
        def _():
            r0 = pl.multiple_of(s * R_MAIN, 8)
            pltpu.sync_copy(
                acc.at[pl.ds(r0, R_MAIN)], out_hbm.at[c, pl.ds(r0, R_MAIN)]
            )

        @pl.when(s == NS - 1)
        def _():
            r0 = (NS - 1) * R_MAIN
            pltpu.sync_copy(
                acc.at[pl.ds(r0, R_LAST)], out_hbm.at[c, pl.ds(r0, R_LAST)]
            )

    return sc_agg


_ROW_BLK = 1000


def _mlp_body(a0_ref, a1_ref, x_ref, wh_ref, bh_ref, wo_ref, bo_ref, o_ref):
    xa = a0_ref[...] + a1_ref[...] - x_ref[...]
    h = (
        jnp.dot(xa, wh_ref[...], preferred_element_type=jnp.float32,
                precision=lax.Precision.HIGHEST)
        + bh_ref[...]
    )
    o = (
        jnp.dot(h, wo_ref[...], preferred_element_type=jnp.float32,
                precision=lax.Precision.HIGHEST)
        + bo_ref[...]
    )
    o_ref[...] = jnp.maximum(o, 0.0)


def _tc_mlp(a0, a1, x, wh, bh, wo, bo):
    grid = (N // _ROW_BLK,)
    row_spec = pl.BlockSpec((_ROW_BLK, D), lambda i: (i, 0))
    full_w = pl.BlockSpec((D, H), lambda i: (0, 0))
    full_b = pl.BlockSpec((1, H), lambda i: (0, 0))
    return pl.pallas_call(
        _mlp_body,
        grid=grid,
        in_specs=[row_spec, row_spec, row_spec, full_w, full_b, full_w, full_b],
        out_specs=pl.BlockSpec((_ROW_BLK, H), lambda i: (i, 0)),
        out_shape=jax.ShapeDtypeStruct((N, H), jnp.float32),
    )(a0, a1, x, wh, bh, wo, bo)


@jax.jit
def kernel(X, ref_a, ref_b, W_hidden, b_hidden, W_out, b_out):
    src = jnp.concatenate([ref_a, ref_b])
    dst = jnp.concatenate([ref_b, ref_a])
    pk = (src | (dst << 16)).reshape(NW, PER_W)
    accs = _sc_aggregate()(X, pk)
    return _tc_mlp(
        accs[0], accs[1], X,
        W_hidden, b_hidden.reshape(1, H), W_out, b_out.reshape(1, H),
    )
